# baseline (speedup 1.0000x reference)
import functools

import jax
import jax.numpy as jnp
from jax import lax
from jax.experimental import pallas as pl
from jax.experimental.pallas import tpu as pltpu
from jax.experimental.pallas import tpu_sc as plsc

VOCAB = 100000
ES = 32
HS = 64
B = 4096
T = 50

_NC = 2
_NS = 16
_NW = _NC * _NS
_ROWS = B * T
_RPW = _ROWS // _NW
_GRP = 128
_NGRP = _RPW // _GRP
_FIRE = 10
_NOUT = _NGRP // _FIRE


def _sc_gather_body(emb_hbm, idx_hbm, out_hbm, idx_v, rows_v, sem):
  wid = lax.axis_index("s") * _NC + lax.axis_index("c")
  pltpu.sync_copy(idx_hbm.at[wid], idx_v)

  def outer(o, carry):
    copies = []
    for j in range(_FIRE):
      cp = pltpu.async_copy(
          emb_hbm.at[idx_v.at[o * _FIRE + j]], rows_v.at[j], sem)
      copies.append(cp)
    for cp in copies:
      cp.wait()
    pltpu.sync_copy(rows_v, out_hbm.at[pl.ds(wid * _NGRP + o * _FIRE, _FIRE)])
    return carry

  lax.fori_loop(0, _NOUT, outer, 0)


_DCH = 640
_DNFC = VOCAB // _DCH
_DTAIL = VOCAB - _DNFC * _DCH


def _sc_depad_body(emb_hbm, out_hbm, buf_v, lin_v):
  wid = lax.axis_index("s") * _NC + lax.axis_index("c")

  def vcopy(nrows):
    def rows(i, carry):
      for u in range(4):
        j = i * 4 + u
        lin_v[pl.ds(j * ES, 16)] = buf_v[j, pl.ds(0, 16)]
        lin_v[pl.ds(j * ES + 16, 16)] = buf_v[j, pl.ds(16, 16)]
      return carry

    lax.fori_loop(0, nrows // 4, rows, 0)

  def do_chunk(c):
    pltpu.sync_copy(emb_hbm.at[pl.ds(c * _DCH, _DCH)], buf_v)
    vcopy(_DCH)
    pltpu.sync_copy(lin_v, out_hbm.at[pl.ds(c * _DCH * ES, _DCH * ES)])

  for k in range(5):
    c = wid + 32 * k
    if k == 4:
      @pl.when(c < _DNFC)
      def _():
        do_chunk(c)
    else:
      do_chunk(c)

  @pl.when(wid < _DTAIL // 8)
  def _():
    r0 = _DNFC * _DCH + wid * 8
    pltpu.sync_copy(emb_hbm.at[pl.ds(r0, 8)], buf_v.at[pl.ds(0, 8)])
    vcopy(8)
    pltpu.sync_copy(lin_v.at[pl.ds(0, 8 * ES)],
                    out_hbm.at[pl.ds(r0 * ES, 8 * ES)])


@functools.cache
def _sc_depad():
  return functools.partial(
      pl.kernel,
      out_type=jax.ShapeDtypeStruct((VOCAB * ES,), jnp.float32),
      mesh=plsc.VectorSubcoreMesh(core_axis_name="c", subcore_axis_name="s"),
      scratch_types=[
          pltpu.VMEM((_DCH, ES), jnp.float32),
          pltpu.VMEM((_DCH * ES,), jnp.float32),
      ],
  )(_sc_depad_body)


@functools.cache
def _sc_gather():
  return functools.partial(
      pl.kernel,
      out_type=jax.ShapeDtypeStruct((_ROWS // _GRP, _GRP, ES), jnp.float32),
      mesh=plsc.VectorSubcoreMesh(core_axis_name="c", subcore_axis_name="s"),
      scratch_types=[
          pltpu.VMEM((_NGRP, _GRP), jnp.int32),
          pltpu.VMEM((_FIRE, _GRP, ES), jnp.float32),
          pltpu.SemaphoreType.DMA,
      ],
      compiler_params=pltpu.CompilerParams(use_tc_tiling_on_sc=False),
  )(_sc_gather_body)


_BB = 512


_F = 4
_FH = _F * HS
_FE = _F * ES
_BQ = B // _F


def _gru_body(e_ref, xf_ref, wrz_ref, win_ref, whn_ref, brz_ref, bin_ref,
              bhn_ref, m_ref, out_ref):
  ecnt = (xf_ref[...] != 0).astype(jnp.float32)
  lfold = jnp.dot(ecnt, m_ref[...],
                  preferred_element_type=jnp.float32).astype(jnp.int32)
  wrz = wrz_ref[...]
  win = win_ref[...]
  whn = whn_ref[...]
  brz = brz_ref[...]
  bin_ = bin_ref[...]
  bhn = bhn_ref[...]

  def step(t, h):
    e_t = e_ref[t]
    eh = jnp.concatenate([e_t, h], axis=1)
    rz = jnp.dot(eh, wrz, preferred_element_type=jnp.float32) + brz
    r = jax.nn.sigmoid(rz[:, :_FH])
    z = jax.nn.sigmoid(rz[:, _FH:])
    gin = jnp.dot(e_t, win, preferred_element_type=jnp.float32) + bin_
    ghn = jnp.dot(h, whn, preferred_element_type=jnp.float32) + bhn
    n = jnp.tanh(gin + r * ghn)
    h_new = (1.0 - z) * n + z * h
    return jnp.where(t < lfold, h_new, h)

  h = lax.fori_loop(0, T, step, jnp.zeros((_BQ, _FH), jnp.float32))
  out_ref[...] = h


def _gru(e4, xf, wrz, win, whn, brz, bin_, bhn, mmat, interpret=False):
  return pl.pallas_call(
      _gru_body,
      grid=(1,),
      in_specs=[
          pl.BlockSpec((T, _BQ, _FE), lambda i: (0, 0, 0)),
          pl.BlockSpec((_BQ, _F * T), lambda i: (0, 0)),
          pl.BlockSpec((_FE + _FH, 2 * _FH), lambda i: (0, 0)),
          pl.BlockSpec((_FE, _FH), lambda i: (0, 0)),
          pl.BlockSpec((_FH, _FH), lambda i: (0, 0)),
          pl.BlockSpec((1, 2 * _FH), lambda i: (0, 0)),
          pl.BlockSpec((1, _FH), lambda i: (0, 0)),
          pl.BlockSpec((1, _FH), lambda i: (0, 0)),
          pl.BlockSpec((_F * T, _FH), lambda i: (0, 0)),
      ],
      out_specs=pl.BlockSpec((_BQ, _FH), lambda i: (0, 0)),
      out_shape=jax.ShapeDtypeStruct((_BQ, _FH), jnp.float32),
      compiler_params=pltpu.CompilerParams(
          dimension_semantics=("arbitrary",),
      ),
      interpret=interpret,
  )(e4, xf, wrz, win, whn, brz, bin_, bhn, mmat)


def kernel(x, emb, w_ih, w_hh, b_ih, b_hh):
  idx3 = x.T.reshape(_NW, _NGRP, _GRP)
  emb_lin = _sc_depad()(emb).reshape(VOCAB, ES)
  e3 = _sc_gather()(emb_lin, idx3)
  e4 = e3.reshape(T, B // _F, _FE)
  xf = x.reshape(B // _F, _F * T)

  eye = jnp.eye(_F, dtype=jnp.float32)
  kr = lambda w: jnp.kron(eye, w)
  wir, wiz, win_ = (w_ih[g * HS:(g + 1) * HS, :].T for g in range(3))
  whr, whz, whn_ = (w_hh[g * HS:(g + 1) * HS, :].T for g in range(3))
  wrz = jnp.concatenate([
      jnp.concatenate([kr(wir), kr(wiz)], axis=1),
      jnp.concatenate([kr(whr), kr(whz)], axis=1),
  ], axis=0)
  win4 = kr(win_)
  whn4 = kr(whn_)
  brz = jnp.concatenate([
      jnp.tile(b_ih[0:HS] + b_hh[0:HS], _F),
      jnp.tile(b_ih[HS:2 * HS] + b_hh[HS:2 * HS], _F),
  ])[None, :]
  bin4 = jnp.tile(b_ih[2 * HS:], _F)[None, :]
  bhn4 = jnp.tile(b_hh[2 * HS:], _F)[None, :]
  kk = jnp.arange(_F * T) // T
  jj = jnp.arange(_FH) // HS
  mmat = (kk[:, None] == jj[None, :]).astype(jnp.float32)

  h4 = _gru(e4, xf, wrz, win4, whn4, brz, bin4, bhn4, mmat)
  return h4.reshape(B, HS)


# speedup vs baseline: 1.1642x; 1.1642x over previous
import functools

import jax
import jax.numpy as jnp
from jax import lax
from jax.experimental import pallas as pl
from jax.experimental.pallas import tpu as pltpu
from jax.experimental.pallas import tpu_sc as plsc

VOCAB = 100000
ES = 32
HS = 64
B = 4096
T = 50

_NC = 2
_NS = 16
_NW = _NC * _NS
_ROWS = B * T
_RPW = _ROWS // _NW
_GRP = 128
_NGRP = _RPW // _GRP
_FIRE = 10
_NOUT = _NGRP // _FIRE


def _sc_gather_body(emb_hbm, idx_hbm, out_hbm, idx_v, rows_v, sem):
  wid = lax.axis_index("s") * _NC + lax.axis_index("c")
  pltpu.sync_copy(idx_hbm.at[wid], idx_v)

  def outer(o, carry):
    copies = []
    for j in range(_FIRE):
      cp = pltpu.async_copy(
          emb_hbm.at[idx_v.at[o * _FIRE + j]], rows_v.at[j], sem)
      copies.append(cp)
    for cp in copies:
      cp.wait()
    pltpu.sync_copy(rows_v, out_hbm.at[pl.ds(wid * _NGRP + o * _FIRE, _FIRE)])
    return carry

  lax.fori_loop(0, _NOUT, outer, 0)


_TCOLS = 8192
_TGRID = -(-VOCAB // _TCOLS)


def _detile_body(et_ref, out_ref):
  out_ref[...] = et_ref[...].T.reshape(_TCOLS * ES // 128, 128)


def _detile(embT, interpret=False):
  return pl.pallas_call(
      _detile_body,
      grid=(_TGRID,),
      in_specs=[pl.BlockSpec((ES, _TCOLS), lambda i: (0, i))],
      out_specs=pl.BlockSpec((_TCOLS * ES // 128, 128), lambda i: (i, 0)),
      out_shape=jax.ShapeDtypeStruct((VOCAB * ES // 128, 128), jnp.float32),
      compiler_params=pltpu.CompilerParams(
          dimension_semantics=("arbitrary",),
      ),
      interpret=interpret,
  )(embT)


@functools.cache
def _sc_gather():
  return functools.partial(
      pl.kernel,
      out_type=jax.ShapeDtypeStruct((_ROWS // _GRP, _GRP, ES), jnp.float32),
      mesh=plsc.VectorSubcoreMesh(core_axis_name="c", subcore_axis_name="s"),
      scratch_types=[
          pltpu.VMEM((_NGRP, _GRP), jnp.int32),
          pltpu.VMEM((_FIRE, _GRP, ES), jnp.float32),
          pltpu.SemaphoreType.DMA,
      ],
      compiler_params=pltpu.CompilerParams(use_tc_tiling_on_sc=False),
  )(_sc_gather_body)


_BB = 512


_F = 4
_FH = _F * HS
_FE = _F * ES
_BQ = B // _F


def _gru_body(e_ref, xf_ref, wrz_ref, win_ref, whn_ref, brz_ref, bin_ref,
              bhn_ref, m_ref, out_ref):
  ecnt = (xf_ref[...] != 0).astype(jnp.float32)
  lfold = jnp.dot(ecnt, m_ref[...],
                  preferred_element_type=jnp.float32).astype(jnp.int32)
  wrz = wrz_ref[...]
  win = win_ref[...]
  whn = whn_ref[...]
  brz = brz_ref[...]
  bin_ = bin_ref[...]
  bhn = bhn_ref[...]

  def step(t, h):
    e_t = e_ref[t]
    eh = jnp.concatenate([e_t, h], axis=1)
    rz = jnp.dot(eh, wrz, preferred_element_type=jnp.float32) + brz
    r = jax.nn.sigmoid(rz[:, :_FH])
    z = jax.nn.sigmoid(rz[:, _FH:])
    gin = jnp.dot(e_t, win, preferred_element_type=jnp.float32) + bin_
    ghn = jnp.dot(h, whn, preferred_element_type=jnp.float32) + bhn
    n = jnp.tanh(gin + r * ghn)
    h_new = (1.0 - z) * n + z * h
    return jnp.where(t < lfold, h_new, h)

  h = lax.fori_loop(0, T, step, jnp.zeros((_BQ, _FH), jnp.float32))
  out_ref[...] = h


def _gru(e4, xf, wrz, win, whn, brz, bin_, bhn, mmat, interpret=False):
  return pl.pallas_call(
      _gru_body,
      grid=(1,),
      in_specs=[
          pl.BlockSpec((T, _BQ, _FE), lambda i: (0, 0, 0)),
          pl.BlockSpec((_BQ, _F * T), lambda i: (0, 0)),
          pl.BlockSpec((_FE + _FH, 2 * _FH), lambda i: (0, 0)),
          pl.BlockSpec((_FE, _FH), lambda i: (0, 0)),
          pl.BlockSpec((_FH, _FH), lambda i: (0, 0)),
          pl.BlockSpec((1, 2 * _FH), lambda i: (0, 0)),
          pl.BlockSpec((1, _FH), lambda i: (0, 0)),
          pl.BlockSpec((1, _FH), lambda i: (0, 0)),
          pl.BlockSpec((_F * T, _FH), lambda i: (0, 0)),
      ],
      out_specs=pl.BlockSpec((_BQ, _FH), lambda i: (0, 0)),
      out_shape=jax.ShapeDtypeStruct((_BQ, _FH), jnp.float32),
      compiler_params=pltpu.CompilerParams(
          dimension_semantics=("arbitrary",),
      ),
      interpret=interpret,
  )(e4, xf, wrz, win, whn, brz, bin_, bhn, mmat)


def kernel(x, emb, w_ih, w_hh, b_ih, b_hh):
  idx3 = x.T.reshape(_NW, _NGRP, _GRP)
  e3 = _sc_gather()(emb, idx3)
  e4 = e3.reshape(T, B // _F, _FE)
  xf = x.reshape(B // _F, _F * T)

  eye = jnp.eye(_F, dtype=jnp.float32)
  kr = lambda w: jnp.kron(eye, w)
  wir, wiz, win_ = (w_ih[g * HS:(g + 1) * HS, :].T for g in range(3))
  whr, whz, whn_ = (w_hh[g * HS:(g + 1) * HS, :].T for g in range(3))
  wrz = jnp.concatenate([
      jnp.concatenate([kr(wir), kr(wiz)], axis=1),
      jnp.concatenate([kr(whr), kr(whz)], axis=1),
  ], axis=0)
  win4 = kr(win_)
  whn4 = kr(whn_)
  brz = jnp.concatenate([
      jnp.tile(b_ih[0:HS] + b_hh[0:HS], _F),
      jnp.tile(b_ih[HS:2 * HS] + b_hh[HS:2 * HS], _F),
  ])[None, :]
  bin4 = jnp.tile(b_ih[2 * HS:], _F)[None, :]
  bhn4 = jnp.tile(b_hh[2 * HS:], _F)[None, :]
  kk = jnp.arange(_F * T) // T
  jj = jnp.arange(_FH) // HS
  mmat = (kk[:, None] == jj[None, :]).astype(jnp.float32)

  h4 = _gru(e4, xf, wrz, win4, whn4, brz, bin4, bhn4, mmat)
  return h4.reshape(B, HS)


# speedup vs baseline: 1.2019x; 1.0324x over previous
import functools

import jax
import jax.numpy as jnp
from jax import lax
from jax.experimental import pallas as pl
from jax.experimental.pallas import tpu as pltpu
from jax.experimental.pallas import tpu_sc as plsc

VOCAB = 100000
ES = 32
HS = 64
B = 4096
T = 50

_NC = 2
_NS = 16
_NW = _NC * _NS
_ROWS = B * T
_RPW = _ROWS // _NW
_GRP = 128
_NGRP = _RPW // _GRP
_FIRE = 10
_NOUT = _NGRP // _FIRE


def _sc_gather_body(emb_hbm, idx_hbm, out_hbm, idx_v, rows_v, sem):
  wid = lax.axis_index("s") * _NC + lax.axis_index("c")
  pltpu.sync_copy(idx_hbm.at[wid], idx_v)

  def outer(o, carry):
    copies = []
    for j in range(_FIRE):
      cp = pltpu.async_copy(
          emb_hbm.at[idx_v.at[o * _FIRE + j]], rows_v.at[j], sem)
      copies.append(cp)
    for cp in copies:
      cp.wait()
    pltpu.sync_copy(rows_v, out_hbm.at[pl.ds(wid * _NGRP + o * _FIRE, _FIRE)])
    return carry

  lax.fori_loop(0, _NOUT, outer, 0)


_TCOLS = 8192
_TGRID = -(-VOCAB // _TCOLS)


def _detile_body(et_ref, out_ref):
  out_ref[...] = et_ref[...].T.reshape(_TCOLS * ES // 128, 128)


def _detile(embT, interpret=False):
  return pl.pallas_call(
      _detile_body,
      grid=(_TGRID,),
      in_specs=[pl.BlockSpec((ES, _TCOLS), lambda i: (0, i))],
      out_specs=pl.BlockSpec((_TCOLS * ES // 128, 128), lambda i: (i, 0)),
      out_shape=jax.ShapeDtypeStruct((VOCAB * ES // 128, 128), jnp.float32),
      compiler_params=pltpu.CompilerParams(
          dimension_semantics=("arbitrary",),
      ),
      interpret=interpret,
  )(embT)


@functools.cache
def _sc_gather():
  return functools.partial(
      pl.kernel,
      out_type=jax.ShapeDtypeStruct((_ROWS // _GRP, _GRP, ES), jnp.float32),
      mesh=plsc.VectorSubcoreMesh(core_axis_name="c", subcore_axis_name="s"),
      scratch_types=[
          pltpu.VMEM((_NGRP, _GRP), jnp.int32),
          pltpu.VMEM((_FIRE, _GRP, ES), jnp.float32),
          pltpu.SemaphoreType.DMA,
      ],
      compiler_params=pltpu.CompilerParams(use_tc_tiling_on_sc=False),
  )(_sc_gather_body)


_BB = 512


_F = 4
_FH = _F * HS
_FE = _F * ES
_BQ = B // _F


_TC = 10
_NTC = T // _TC


def _gru_body(e_ref, xf_ref, wi_ref, wh_ref, gib_ref, bhn_ref, m_ref,
              out_ref, gi_s, h_s, lf_s):
  i = pl.program_id(0)

  @pl.when(i == 0)
  def _():
    ecnt = (xf_ref[...] != 0).astype(jnp.float32)
    lf_s[...] = jnp.dot(ecnt, m_ref[...],
                        preferred_element_type=jnp.float32).astype(jnp.int32)
    h_s[...] = jnp.zeros((_BQ, _FH), jnp.float32)

  e2d = e_ref[...].reshape(_TC * _BQ, _FE)
  gi_s[...] = (jnp.dot(e2d, wi_ref[...], preferred_element_type=jnp.float32)
               + gib_ref[...]).reshape(_TC, _BQ, 3 * _FH)
  wh = wh_ref[...]
  bhn = bhn_ref[...]
  lfold = lf_s[...]
  t0 = i * _TC

  def step(k, h):
    gi = gi_s[k]
    gh = jnp.dot(h, wh, preferred_element_type=jnp.float32)
    r = jax.nn.sigmoid(gi[:, :_FH] + gh[:, :_FH])
    z = jax.nn.sigmoid(gi[:, _FH:2 * _FH] + gh[:, _FH:2 * _FH])
    n = jnp.tanh(gi[:, 2 * _FH:] + r * (gh[:, 2 * _FH:] + bhn))
    h_new = (1.0 - z) * n + z * h
    return jnp.where(t0 + k < lfold, h_new, h)

  h = lax.fori_loop(0, _TC, step, h_s[...])
  h_s[...] = h

  @pl.when(i == _NTC - 1)
  def _():
    out_ref[...] = h


def _gru(e4, xf, wi4, wh4, gib4, bhn4, mmat, interpret=False):
  return pl.pallas_call(
      _gru_body,
      grid=(_NTC,),
      in_specs=[
          pl.BlockSpec((_TC, _BQ, _FE), lambda i: (i, 0, 0)),
          pl.BlockSpec((_BQ, _F * T), lambda i: (0, 0)),
          pl.BlockSpec((_FE, 3 * _FH), lambda i: (0, 0)),
          pl.BlockSpec((_FH, 3 * _FH), lambda i: (0, 0)),
          pl.BlockSpec((1, 3 * _FH), lambda i: (0, 0)),
          pl.BlockSpec((1, _FH), lambda i: (0, 0)),
          pl.BlockSpec((_F * T, _FH), lambda i: (0, 0)),
      ],
      out_specs=pl.BlockSpec((_BQ, _FH), lambda i: (0, 0)),
      out_shape=jax.ShapeDtypeStruct((_BQ, _FH), jnp.float32),
      scratch_shapes=[
          pltpu.VMEM((_TC, _BQ, 3 * _FH), jnp.float32),
          pltpu.VMEM((_BQ, _FH), jnp.float32),
          pltpu.VMEM((_BQ, _FH), jnp.int32),
      ],
      compiler_params=pltpu.CompilerParams(
          dimension_semantics=("arbitrary",),
      ),
      interpret=interpret,
  )(e4, xf, wi4, wh4, gib4, bhn4, mmat)


def kernel(x, emb, w_ih, w_hh, b_ih, b_hh):
  idx3 = x.T.reshape(_NW, _NGRP, _GRP)
  e3 = _sc_gather()(emb, idx3)
  e4 = e3.reshape(T, B // _F, _FE)
  xf = x.reshape(B // _F, _F * T)

  eye = jnp.eye(_F, dtype=jnp.float32)
  kr = lambda w: jnp.kron(eye, w)
  wi4 = jnp.concatenate(
      [kr(w_ih[g * HS:(g + 1) * HS, :].T) for g in range(3)],
      axis=1)
  wh4 = jnp.concatenate(
      [kr(w_hh[g * HS:(g + 1) * HS, :].T) for g in range(3)],
      axis=1)
  gib4 = jnp.concatenate([
      jnp.tile(b_ih[0:HS] + b_hh[0:HS], _F),
      jnp.tile(b_ih[HS:2 * HS] + b_hh[HS:2 * HS], _F),
      jnp.tile(b_ih[2 * HS:], _F),
  ])[None, :]
  bhn4 = jnp.tile(b_hh[2 * HS:], _F)[None, :]
  kk = jnp.arange(_F * T) // T
  jj = jnp.arange(_FH) // HS
  mmat = (kk[:, None] == jj[None, :]).astype(jnp.float32)

  h4 = _gru(e4, xf, wi4, wh4, gib4, bhn4, mmat)
  return h4.reshape(B, HS)
